# Initial kernel scaffold; baseline (speedup 1.0000x reference)
import jax
import jax.numpy as jnp
from jax.experimental import pallas as pl


def kernel(x_curr, u_curr, logP0, W):
    raise NotImplementedError("write your pallas kernel here")


# speedup vs baseline: 6.7947x; 6.7947x over previous
import functools

import jax
import jax.numpy as jnp
from jax.experimental import pallas as pl
from jax.experimental.pallas import tpu as pltpu


def _body(x_ref, u_ref, wm_ref, lp_ref, o_ref, *, TB, N, U):
    x = x_ref[...]
    u = u_ref[...]
    c = jax.lax.broadcasted_iota(jnp.int32, (TB, N * U), 1)
    mask = (c // U) == x
    u_t = jnp.tile(u, (1, N))
    z = jnp.where(mask, u_t, 0.0).astype(jnp.bfloat16)
    stim = jnp.dot(z, wm_ref[...], preferred_element_type=jnp.float32)
    onehot = (jax.lax.broadcasted_iota(jnp.int32, (TB, N), 1) == x).astype(jnp.float32)
    base = jnp.dot(onehot, lp_ref[...], preferred_element_type=jnp.float32)
    logits = base + stim
    m = jnp.max(logits, axis=1, keepdims=True)
    ex = jnp.exp(logits - m)
    lz = jnp.log(jnp.sum(ex, axis=1, keepdims=True)) + m
    o_ref[...] = logits - lz


@functools.partial(jax.jit, static_argnames=("interpret",))
def kernel(x_curr, u_curr, logP0, W, interpret=False):
    T = x_curr.shape[0]
    N = logP0.shape[0]
    U = u_curr.shape[1]
    cols = jnp.arange(N)[None, :]
    srows = jnp.arange(N)[:, None]
    k = jnp.clip(cols - (cols > srows).astype(jnp.int32), 0, N - 2)
    W64 = jnp.take_along_axis(W, k[:, :, None], axis=1)
    W64 = jnp.where((cols == srows)[:, :, None], 0.0, W64)
    Wm = W64.transpose(0, 2, 1).reshape(N * U, N).astype(jnp.bfloat16)

    TB = 512
    x2 = x_curr.astype(jnp.int32).reshape(T, 1)
    grid = (T // TB,)
    out = pl.pallas_call(
        functools.partial(_body, TB=TB, N=N, U=U),
        grid=grid,
        in_specs=[
            pl.BlockSpec((TB, 1), lambda i: (i, 0)),
            pl.BlockSpec((TB, U), lambda i: (i, 0)),
            pl.BlockSpec((N * U, N), lambda i: (0, 0)),
            pl.BlockSpec((N, N), lambda i: (0, 0)),
        ],
        out_specs=pl.BlockSpec((TB, N), lambda i: (i, 0)),
        out_shape=jax.ShapeDtypeStruct((T, N), jnp.float32),
        compiler_params=pltpu.CompilerParams(
            dimension_semantics=("arbitrary",),
        ),
        interpret=interpret,
    )(x2, u_curr, Wm, logP0)
    return out
